# baseline (speedup 1.0000x reference)
import dataclasses
import functools

import jax
import jax.numpy as jnp
from jax import lax
from jax.experimental import pallas as pl
from jax.experimental.pallas import tpu as pltpu
from jax.experimental.pallas import tpu_sc as plsc

N = 10000
F = 128
E = 320000
EPW = E // 32
CHUNK = 32
NCHUNK = 320
E_PAD = 32 * NCHUNK * CHUNK
IBLK = 32
NIBLK = NCHUNK // IBLK
NRING = 8
ACC_PAD = 10112
STRIPE = ACC_PAD // 16
DEG_PAD = 10240
DSTRIPE = DEG_PAD // 16
ROWB = 10
RB = N // ROWB

_mesh = plsc.VectorSubcoreMesh(core_axis_name="c", subcore_axis_name="s")

_sc_params = pltpu.CompilerParams()
if "needs_layout_passes" in pltpu.CompilerParams.__dataclass_fields__:
    _sc_params = dataclasses.replace(_sc_params, needs_layout_passes=False)


@functools.partial(
    pl.kernel,
    out_type=jax.ShapeDtypeStruct((DEG_PAD,), jnp.float32),
    mesh=_mesh,
    scratch_types=[
        pltpu.VMEM((EPW * 2,), jnp.int32),
        pltpu.VMEM((DEG_PAD,), jnp.float32),
        pltpu.VMEM((DSTRIPE,), jnp.float32),
        pltpu.VMEM((DSTRIPE,), jnp.float32),
        pltpu.VMEM_SHARED((16, DEG_PAD), jnp.float32),
    ],
    compiler_params=_sc_params,
)
def _deg_kernel(e_hbm, deg_hbm, idx_v, hist_v, buf_v, acc_v, stage_sh):
    c = lax.axis_index("c")
    s = lax.axis_index("s")

    @pl.when(c == 0)
    def _count():
        zeros = jnp.zeros((16,), jnp.float32)

        @pl.loop(0, DEG_PAD, step=16)
        def _(i):
            hist_v[pl.ds(i, 16)] = zeros

        pltpu.sync_copy(e_hbm.at[pl.ds(E + s * (EPW * 2), EPW * 2)], idx_v)
        ones = jnp.ones((16,), jnp.float32)

        @pl.loop(0, EPW * 2, step=16)
        def _(i):
            plsc.addupdate_scatter(hist_v, [idx_v[pl.ds(i, 16)]], ones)

        pltpu.sync_copy(hist_v, stage_sh.at[s])

    plsc.subcore_barrier()

    @pl.when(c == 0)
    def _merge():
        ones = jnp.ones((16,), jnp.float32)

        @pl.loop(0, DSTRIPE, step=16)
        def _(k):
            acc_v[pl.ds(k, 16)] = ones

        @pl.loop(0, 16)
        def _(t):
            pltpu.sync_copy(stage_sh.at[t, pl.ds(s * DSTRIPE, DSTRIPE)], buf_v)

            @pl.loop(0, DSTRIPE, step=16)
            def _(k):
                acc_v[pl.ds(k, 16)] = acc_v[pl.ds(k, 16)] + buf_v[pl.ds(k, 16)]

        pltpu.sync_copy(acc_v, deg_hbm.at[pl.ds(s * DSTRIPE, DSTRIPE)])


@functools.partial(
    pl.kernel,
    out_type=jax.ShapeDtypeStruct((2, ACC_PAD, F), jnp.float32),
    mesh=_mesh,
    scratch_types=[
        pltpu.VMEM((IBLK, CHUNK), jnp.int32),
        pltpu.VMEM((IBLK, CHUNK), jnp.int32),
        *[pltpu.VMEM((CHUNK, F), jnp.float32) for _ in range(NRING)],
        pltpu.VMEM_SHARED((ACC_PAD, F), jnp.float32),
        *[pltpu.SemaphoreType.DMA for _ in range(2 * NRING)],
    ],
)
def _agg_kernel(y_hbm, src_hbm, dst_hbm, out_hbm, src_v, dst_v, *rest):
    rows = rest[:NRING]
    acc_sh = rest[NRING]
    gsems = rest[NRING + 1:2 * NRING + 1]
    ssems = rest[2 * NRING + 1:]
    row0 = rows[0]
    c = lax.axis_index("c")
    s = lax.axis_index("s")
    wid = c * 16 + s
    zeros = jnp.zeros((16,), jnp.float32)

    @pl.loop(0, CHUNK)
    def _(r):
        @pl.loop(0, F, step=16)
        def _(k):
            row0[r, pl.ds(k, 16)] = zeros

    @pl.loop(0, STRIPE // CHUNK)
    def _(k):
        pltpu.sync_copy(row0, acc_sh.at[pl.ds(s * STRIPE + k * CHUNK, CHUNK)])

    pltpu.sync_copy(row0.at[pl.ds(0, STRIPE % CHUNK)],
                    acc_sh.at[pl.ds(s * STRIPE + STRIPE - STRIPE % CHUNK,
                                    STRIPE % CHUNK)])
    plsc.subcore_barrier()

    def _g_start(j, b):
        pltpu.async_copy(y_hbm.at[src_v.at[j]], rows[b], gsems[b])

    def _g_wait(b):
        pltpu.make_async_copy(y_hbm.at[src_v.at[0]], rows[b], gsems[b]).wait()

    def _s_start(j, b):
        pltpu.async_copy(rows[b], acc_sh.at[dst_v.at[j]], ssems[b], add=True)

    def _s_wait(b):
        pltpu.make_async_copy(rows[b], acc_sh.at[dst_v.at[0]], ssems[b]).wait()

    @pl.loop(0, NIBLK)
    def _(blk):
        pltpu.sync_copy(src_hbm.at[wid * NIBLK + blk], src_v)
        pltpu.sync_copy(dst_hbm.at[wid * NIBLK + blk], dst_v)

        for b in range(NRING):
            _g_start(b, b)

        @pl.loop(0, IBLK - NRING, step=NRING)
        def _(j):
            for b in range(NRING):
                _g_wait(b)
                _s_start(j + b, b)
            for b in range(NRING):
                _s_wait(b)
                _g_start(j + NRING + b, b)

        for b in range(NRING):
            _g_wait(b)
            _s_start(IBLK - NRING + b, b)
        for b in range(NRING):
            _s_wait(b)

    plsc.subcore_barrier()
    pltpu.sync_copy(acc_sh.at[pl.ds(s * STRIPE, STRIPE)],
                    out_hbm.at[c, pl.ds(s * STRIPE, STRIPE)])


def _t0_body(x_ref, w_ref, o_ref):
    o_ref[...] = jnp.dot(x_ref[...], w_ref[...],
                         preferred_element_type=jnp.float32)


def _t1_body(xw_ref, deg_ref, o_ref):
    o_ref[...] = lax.rsqrt(deg_ref[...]) * xw_ref[...]


def _t2_body(p_ref, y_ref, deg_ref, b_ref, w_ref, o_ref):
    dis = lax.rsqrt(deg_ref[...])
    agg = p_ref[0] + p_ref[1] + y_ref[...]
    h = jnp.maximum(dis * agg + b_ref[...], 0.0)
    hw = jnp.dot(h, w_ref[...], preferred_element_type=jnp.float32)
    o_ref[...] = dis * hw


def _t3_body(p_ref, y_ref, deg_ref, b_ref, o_ref):
    dis = lax.rsqrt(deg_ref[...])
    o_ref[...] = dis * (p_ref[0] + p_ref[1] + y_ref[...]) + b_ref[...]


_row_spec = pl.BlockSpec((RB, F), lambda i: (i, 0))
_deg_spec = pl.BlockSpec((RB, 1), lambda i: (i, 0))
_w_spec = pl.BlockSpec((F, F), lambda i: (0, 0))
_b_spec = pl.BlockSpec((1, F), lambda i: (0, 0))
_p_spec = pl.BlockSpec((2, RB, F), lambda i: (0, i, 0))
_out_sds = jax.ShapeDtypeStruct((N, F), jnp.float32)

_t0 = pl.pallas_call(
    _t0_body, grid=(ROWB,), out_shape=_out_sds,
    in_specs=[_row_spec, _w_spec], out_specs=_row_spec)
_t1 = pl.pallas_call(
    _t1_body, grid=(ROWB,), out_shape=_out_sds,
    in_specs=[_row_spec, _deg_spec], out_specs=_row_spec)
_t2 = pl.pallas_call(
    _t2_body, grid=(ROWB,), out_shape=_out_sds,
    in_specs=[_p_spec, _row_spec, _deg_spec, _b_spec, _w_spec],
    out_specs=_row_spec)
_t3 = pl.pallas_call(
    _t3_body, grid=(ROWB,), out_shape=_out_sds,
    in_specs=[_p_spec, _row_spec, _deg_spec, _b_spec], out_specs=_row_spec)


def kernel(x, edge_index, W1, b1, W2, b2):
    e32 = edge_index.astype(jnp.int32)
    npad = E_PAD - E
    pad_ar = jnp.arange(npad, dtype=jnp.int32)
    src = jnp.concatenate([e32[0], pad_ar % N]
                          ).reshape(32 * NIBLK, IBLK, CHUNK)
    dst = jnp.concatenate([e32[1], N + pad_ar % (ACC_PAD - N)]
                          ).reshape(32 * NIBLK, IBLK, CHUNK)
    b1r = b1.reshape(1, F)
    b2r = b2.reshape(1, F)

    deg = _deg_kernel(e32.reshape(2 * E)).reshape(DEG_PAD, 1)
    xw1 = _t0(x, W1)
    y1 = _t1(xw1, deg)
    p1 = _agg_kernel(y1, src, dst)
    y2 = _t2(p1, y1, deg, b1r, W2)
    p2 = _agg_kernel(y2, src, dst)
    return _t3(p2, y2, deg, b2r)


# speedup vs baseline: 1.0861x; 1.0861x over previous
import dataclasses
import functools

import jax
import jax.numpy as jnp
from jax import lax
from jax.experimental import pallas as pl
from jax.experimental.pallas import tpu as pltpu
from jax.experimental.pallas import tpu_sc as plsc

N = 10000
F = 128
E = 320000
EPW = E // 32
CHUNK = 64
NCHUNK = 160
E_PAD = 32 * NCHUNK * CHUNK
IBLK = 32
NIBLK = NCHUNK // IBLK
NRING = 4
ACC_PAD = 10112
STRIPE = ACC_PAD // 16
DEG_PAD = 10240
DSTRIPE = DEG_PAD // 16
ROWB = 5
RB = N // ROWB

_mesh = plsc.VectorSubcoreMesh(core_axis_name="c", subcore_axis_name="s")

_sc_params = pltpu.CompilerParams()
if "needs_layout_passes" in pltpu.CompilerParams.__dataclass_fields__:
    _sc_params = dataclasses.replace(_sc_params, needs_layout_passes=False)


@functools.partial(
    pl.kernel,
    out_type=jax.ShapeDtypeStruct((DEG_PAD,), jnp.float32),
    mesh=_mesh,
    scratch_types=[
        pltpu.VMEM((EPW * 2,), jnp.int32),
        pltpu.VMEM((DEG_PAD,), jnp.float32),
        pltpu.VMEM((DSTRIPE,), jnp.float32),
        pltpu.VMEM((DSTRIPE,), jnp.float32),
        pltpu.VMEM_SHARED((16, DEG_PAD), jnp.float32),
    ],
    compiler_params=_sc_params,
)
def _deg_kernel(e_hbm, deg_hbm, idx_v, hist_v, buf_v, acc_v, stage_sh):
    c = lax.axis_index("c")
    s = lax.axis_index("s")

    @pl.when(c == 0)
    def _count():
        zeros = jnp.zeros((16,), jnp.float32)

        @pl.loop(0, DEG_PAD, step=16)
        def _(i):
            hist_v[pl.ds(i, 16)] = zeros

        pltpu.sync_copy(e_hbm.at[pl.ds(E + s * (EPW * 2), EPW * 2)], idx_v)
        ones = jnp.ones((16,), jnp.float32)

        @pl.loop(0, EPW * 2, step=16)
        def _(i):
            plsc.addupdate_scatter(hist_v, [idx_v[pl.ds(i, 16)]], ones)

        pltpu.sync_copy(hist_v, stage_sh.at[s])

    plsc.subcore_barrier()

    @pl.when(c == 0)
    def _merge():
        ones = jnp.ones((16,), jnp.float32)

        @pl.loop(0, DSTRIPE, step=16)
        def _(k):
            acc_v[pl.ds(k, 16)] = ones

        @pl.loop(0, 16)
        def _(t):
            pltpu.sync_copy(stage_sh.at[t, pl.ds(s * DSTRIPE, DSTRIPE)], buf_v)

            @pl.loop(0, DSTRIPE, step=16)
            def _(k):
                acc_v[pl.ds(k, 16)] = acc_v[pl.ds(k, 16)] + buf_v[pl.ds(k, 16)]

        pltpu.sync_copy(acc_v, deg_hbm.at[pl.ds(s * DSTRIPE, DSTRIPE)])


@functools.partial(
    pl.kernel,
    out_type=jax.ShapeDtypeStruct((2, ACC_PAD, F), jnp.float32),
    mesh=_mesh,
    scratch_types=[
        pltpu.VMEM((IBLK, CHUNK), jnp.int32),
        pltpu.VMEM((IBLK, CHUNK), jnp.int32),
        *[pltpu.VMEM((CHUNK, F), jnp.float32) for _ in range(NRING)],
        pltpu.VMEM_SHARED((ACC_PAD, F), jnp.float32),
        *[pltpu.SemaphoreType.DMA for _ in range(2 * NRING)],
    ],
)
def _agg_kernel(y_hbm, src_hbm, dst_hbm, out_hbm, src_v, dst_v, *rest):
    rows = rest[:NRING]
    acc_sh = rest[NRING]
    gsems = rest[NRING + 1:2 * NRING + 1]
    ssems = rest[2 * NRING + 1:]
    row0 = rows[0]
    c = lax.axis_index("c")
    s = lax.axis_index("s")
    wid = c * 16 + s
    zeros = jnp.zeros((16,), jnp.float32)

    @pl.loop(0, CHUNK)
    def _(r):
        @pl.loop(0, F, step=16)
        def _(k):
            row0[r, pl.ds(k, 16)] = zeros

    @pl.loop(0, STRIPE // CHUNK)
    def _(k):
        pltpu.sync_copy(row0, acc_sh.at[pl.ds(s * STRIPE + k * CHUNK, CHUNK)])

    pltpu.sync_copy(row0.at[pl.ds(0, STRIPE % CHUNK)],
                    acc_sh.at[pl.ds(s * STRIPE + STRIPE - STRIPE % CHUNK,
                                    STRIPE % CHUNK)])
    plsc.subcore_barrier()

    def _g_start(j, b):
        pltpu.async_copy(y_hbm.at[src_v.at[j]], rows[b], gsems[b])

    def _g_wait(b):
        pltpu.make_async_copy(y_hbm.at[src_v.at[0]], rows[b], gsems[b]).wait()

    def _s_start(j, b):
        pltpu.async_copy(rows[b], acc_sh.at[dst_v.at[j]], ssems[b], add=True)

    def _s_wait(b):
        pltpu.make_async_copy(rows[b], acc_sh.at[dst_v.at[0]], ssems[b]).wait()

    @pl.loop(0, NIBLK)
    def _(blk):
        pltpu.sync_copy(src_hbm.at[wid * NIBLK + blk], src_v)
        pltpu.sync_copy(dst_hbm.at[wid * NIBLK + blk], dst_v)

        for b in range(NRING):
            _g_start(b, b)

        @pl.loop(0, IBLK - NRING, step=NRING)
        def _(j):
            for b in range(NRING):
                _g_wait(b)
                _s_start(j + b, b)
            for b in range(NRING):
                _s_wait(b)
                _g_start(j + NRING + b, b)

        for b in range(NRING):
            _g_wait(b)
            _s_start(IBLK - NRING + b, b)
        for b in range(NRING):
            _s_wait(b)

    plsc.subcore_barrier()
    pltpu.sync_copy(acc_sh.at[pl.ds(s * STRIPE, STRIPE)],
                    out_hbm.at[c, pl.ds(s * STRIPE, STRIPE)])


def _t0_body(x_ref, w_ref, o_ref):
    o_ref[...] = jnp.dot(x_ref[...], w_ref[...],
                         preferred_element_type=jnp.float32)


def _t1_body(xw_ref, deg_ref, o_ref):
    o_ref[...] = lax.rsqrt(deg_ref[...]) * xw_ref[...]


def _t2_body(p_ref, y_ref, deg_ref, b_ref, w_ref, o_ref):
    dis = lax.rsqrt(deg_ref[...])
    agg = p_ref[0] + p_ref[1] + y_ref[...]
    h = jnp.maximum(dis * agg + b_ref[...], 0.0)
    hw = jnp.dot(h, w_ref[...], preferred_element_type=jnp.float32)
    o_ref[...] = dis * hw


def _t3_body(p_ref, y_ref, deg_ref, b_ref, o_ref):
    dis = lax.rsqrt(deg_ref[...])
    o_ref[...] = dis * (p_ref[0] + p_ref[1] + y_ref[...]) + b_ref[...]


_row_spec = pl.BlockSpec((RB, F), lambda i: (i, 0))
_deg_spec = pl.BlockSpec((RB, 1), lambda i: (i, 0))
_w_spec = pl.BlockSpec((F, F), lambda i: (0, 0))
_b_spec = pl.BlockSpec((1, F), lambda i: (0, 0))
_p_spec = pl.BlockSpec((2, RB, F), lambda i: (0, i, 0))
_out_sds = jax.ShapeDtypeStruct((N, F), jnp.float32)

_t0 = pl.pallas_call(
    _t0_body, grid=(ROWB,), out_shape=_out_sds,
    in_specs=[_row_spec, _w_spec], out_specs=_row_spec)
_t1 = pl.pallas_call(
    _t1_body, grid=(ROWB,), out_shape=_out_sds,
    in_specs=[_row_spec, _deg_spec], out_specs=_row_spec)
_t2 = pl.pallas_call(
    _t2_body, grid=(ROWB,), out_shape=_out_sds,
    in_specs=[_p_spec, _row_spec, _deg_spec, _b_spec, _w_spec],
    out_specs=_row_spec)
_t3 = pl.pallas_call(
    _t3_body, grid=(ROWB,), out_shape=_out_sds,
    in_specs=[_p_spec, _row_spec, _deg_spec, _b_spec], out_specs=_row_spec)


def kernel(x, edge_index, W1, b1, W2, b2):
    e32 = edge_index.astype(jnp.int32)
    npad = E_PAD - E
    pad_ar = jnp.arange(npad, dtype=jnp.int32)
    src = jnp.concatenate([e32[0], pad_ar % N]
                          ).reshape(32 * NIBLK, IBLK, CHUNK)
    dst = jnp.concatenate([e32[1], N + pad_ar % (ACC_PAD - N)]
                          ).reshape(32 * NIBLK, IBLK, CHUNK)
    b1r = b1.reshape(1, F)
    b2r = b2.reshape(1, F)

    deg = _deg_kernel(e32.reshape(2 * E)).reshape(DEG_PAD, 1)
    xw1 = _t0(x, W1)
    y1 = _t1(xw1, deg)
    p1 = _agg_kernel(y1, src, dst)
    y2 = _t2(p1, y1, deg, b1r, W2)
    p2 = _agg_kernel(y2, src, dst)
    return _t3(p2, y2, deg, b2r)
